# baseline (speedup 1.0000x reference)
import functools

import jax
import jax.numpy as jnp
from jax import lax
from jax.experimental import pallas as pl
from jax.experimental.pallas import tpu as pltpu
from jax.experimental.pallas import tpu_sc as plsc

DIM = 32
NUM_HEADS = 8
NFEAT = NUM_HEADS * DIM * DIM
NPOS = 4096
RSLAB = 4
UNROLL = 8


def _sc_gather(tableT, mapping):
    nb, seq = mapping.shape
    info = plsc.get_sparse_core_info()
    nc, ns = info.num_cores, info.num_subcores
    nw = nc * ns
    f_per_w = NFEAT // nw
    n_groups = f_per_w // RSLAB
    n_vec = seq // 16

    mesh = plsc.VectorSubcoreMesh(core_axis_name="c", subcore_axis_name="s")

    @functools.partial(
        pl.kernel,
        mesh=mesh,
        compiler_params=pltpu.CompilerParams(needs_layout_passes=False),
        out_type=jax.ShapeDtypeStruct((nb, NFEAT, seq), jnp.float32),
        scratch_types=[
            pltpu.VMEM((nb, seq), jnp.int32),
            pltpu.VMEM((RSLAB, NPOS), jnp.float32),
            pltpu.VMEM((RSLAB, NPOS), jnp.float32),
            pltpu.VMEM((nb, RSLAB, seq), jnp.float32),
            pltpu.VMEM((nb, RSLAB, seq), jnp.float32),
            pltpu.SemaphoreType.DMA,
            pltpu.SemaphoreType.DMA,
            pltpu.SemaphoreType.DMA,
            pltpu.SemaphoreType.DMA,
        ],
    )
    def k(table_hbm, idx_hbm, out_hbm, idx_v,
          slab0, slab1, ob0, ob1, i0, i1, o0, o1):
        wid = lax.axis_index("s") * nc + lax.axis_index("c")
        base_f = wid * f_per_w
        pltpu.sync_copy(idx_hbm, idx_v)

        slabs = (slab0, slab1)
        obufs = (ob0, ob1)
        isems = (i0, i1)
        osems = (o0, o1)

        def in_start(g, p):
            pltpu.make_async_copy(
                table_hbm.at[pl.ds(base_f + g * RSLAB, RSLAB)],
                slabs[p], isems[p]).start()

        def in_wait(p):
            pltpu.make_async_copy(
                table_hbm.at[pl.ds(base_f, RSLAB)], slabs[p], isems[p]).wait()

        def out_start(g, p):
            for b in range(nb):
                pltpu.make_async_copy(
                    obufs[p].at[pl.ds(b, 1)],
                    out_hbm.at[pl.ds(b, 1), pl.ds(base_f + g * RSLAB, RSLAB)],
                    osems[p]).start()

        def out_drain(p):
            for b in range(nb):
                pltpu.make_async_copy(
                    obufs[p].at[pl.ds(b, 1)],
                    out_hbm.at[pl.ds(0, 1), pl.ds(base_f, RSLAB)],
                    osems[p]).wait()

        row_ids = tuple(
            jnp.full((16,), r, dtype=jnp.int32) for r in range(RSLAB))

        def compute(p):
            slab = slabs[p]
            obuf = obufs[p]

            def body(vb, carry):
                off0 = vb * (16 * UNROLL)
                for u2 in range(UNROLL):
                    off = off0 + u2 * 16
                    for b in range(nb):
                        idxv = idx_v[b, pl.ds(off, 16)]
                        for r in range(RSLAB):
                            obuf[b, r, pl.ds(off, 16)] = plsc.load_gather(
                                slab, [row_ids[r], idxv])
                return carry

            lax.fori_loop(0, n_vec // UNROLL, body, 0)

        in_start(0, 0)
        in_start(1, 1)
        in_wait(0)
        compute(0)
        out_start(0, 0)
        in_start(2, 0)
        in_wait(1)
        compute(1)
        out_start(1, 1)
        in_start(3, 1)

        def loop_body(t, carry):
            for u in range(2):
                g = 2 * t + 2 + u
                in_wait(u)
                out_drain(u)
                compute(u)
                out_start(g, u)
                in_start(g + 2, u)
            return carry

        lax.fori_loop(0, (n_groups - 4) // 2, loop_body, 0)

        for u in range(2):
            in_wait(u)
            out_drain(u)
            compute(u)
            out_start(n_groups - 2 + u, u)
        out_drain(0)
        out_drain(1)

    return k


def kernel(mapping, maps):
    tableT = maps.transpose(1, 2, 3, 0).reshape(NFEAT, NPOS)
    idx = mapping.astype(jnp.int32)
    out = _sc_gather(tableT, idx)(tableT, idx)
    nb, seq = mapping.shape
    return out.reshape(nb, NUM_HEADS, DIM, DIM, seq).transpose(0, 4, 1, 2, 3)
